# Initial kernel scaffold; baseline (speedup 1.0000x reference)
import jax
import jax.numpy as jnp
from jax.experimental import pallas as pl


def kernel(x, evolutionary_distance, W1, a_src1, a_dst1, b1, W2, a_src2, a_dst2, b2, We, be, Wf1, bf1, Wf2, bf2, edge_index, batch):
    raise NotImplementedError("write your pallas kernel here")


# speedup vs baseline: 2.3281x; 2.3281x over previous
import jax
import jax.numpy as jnp
from jax.experimental import pallas as pl

_N = 94208
_E = 524288
_G = 2048
_H = 4
_C = 64
_BN = 512


def _k1_body(x_ref, w1_ref, as_ref, ad_ref, s_ref, d_ref):
    x = x_ref[...]
    w1 = w1_ref[...]
    h = jnp.dot(x, w1, preferred_element_type=jnp.float32)
    h4 = h.reshape(_BN, _H, _C)
    s_ref[...] = jnp.sum(h4 * as_ref[...][None], axis=-1)
    d_ref[...] = jnp.sum(h4 * ad_ref[...][None], axis=-1)


def _k1(x, W1, a_src1, a_dst1):
    grid = _N // _BN
    return pl.pallas_call(
        _k1_body,
        grid=(grid,),
        in_specs=[
            pl.BlockSpec((_BN, 4), lambda i: (i, 0)),
            pl.BlockSpec((4, _H * _C), lambda i: (0, 0)),
            pl.BlockSpec((_H, _C), lambda i: (0, 0)),
            pl.BlockSpec((_H, _C), lambda i: (0, 0)),
        ],
        out_specs=[
            pl.BlockSpec((_BN, _H), lambda i: (i, 0)),
            pl.BlockSpec((_BN, _H), lambda i: (i, 0)),
        ],
        out_shape=[
            jax.ShapeDtypeStruct((_N, _H), jnp.float32),
            jax.ShapeDtypeStruct((_N, _H), jnp.float32),
        ],
    )(x, W1, a_src1, a_dst1)


def _k2_body(zx_ref, den_ref, w1_ref, b1_ref, w2_ref, as2_ref, ad2_ref,
             p_ref, s2_ref, d2_ref):
    zx = zx_ref[...]
    den = den_ref[...]
    w1 = w1_ref[...]
    outs = []
    for h in range(_H):
        zh = zx[:, h * 4:(h + 1) * 4] / (den[:, h:h + 1] + 1e-16)
        outs.append(jnp.dot(zh, w1[:, h * _C:(h + 1) * _C],
                            preferred_element_type=jnp.float32))
    h1 = jnp.concatenate(outs, axis=1) + b1_ref[...][None, :]
    g = jnp.where(h1 > 0, h1, jnp.exp(jnp.minimum(h1, 0.0)) - 1.0)
    p = jnp.dot(g, w2_ref[...], preferred_element_type=jnp.float32)
    p_ref[...] = p
    s2_ref[...] = jnp.sum(p * as2_ref[...], axis=-1, keepdims=True)
    d2_ref[...] = jnp.sum(p * ad2_ref[...], axis=-1, keepdims=True)


def _k2(zx, den, W1, b1, W2, a_src2, a_dst2):
    grid = _N // _BN
    return pl.pallas_call(
        _k2_body,
        grid=(grid,),
        in_specs=[
            pl.BlockSpec((_BN, _H * 4), lambda i: (i, 0)),
            pl.BlockSpec((_BN, _H), lambda i: (i, 0)),
            pl.BlockSpec((4, _H * _C), lambda i: (0, 0)),
            pl.BlockSpec((_H * _C,), lambda i: (0,)),
            pl.BlockSpec((_H * _C, _C), lambda i: (0, 0)),
            pl.BlockSpec((1, _C), lambda i: (0, 0)),
            pl.BlockSpec((1, _C), lambda i: (0, 0)),
        ],
        out_specs=[
            pl.BlockSpec((_BN, _C), lambda i: (i, 0)),
            pl.BlockSpec((_BN, 1), lambda i: (i, 0)),
            pl.BlockSpec((_BN, 1), lambda i: (i, 0)),
        ],
        out_shape=[
            jax.ShapeDtypeStruct((_N, _C), jnp.float32),
            jax.ShapeDtypeStruct((_N, 1), jnp.float32),
            jax.ShapeDtypeStruct((_N, 1), jnp.float32),
        ],
    )(zx, den, W1, b1, W2, a_src2, a_dst2)


def _k3_body(zp_ref, den2_ref, b2_ref, batch_ref, sums_ref, cnt_ref):
    i = pl.program_id(0)

    @pl.when(i == 0)
    def _init():
        sums_ref[...] = jnp.zeros_like(sums_ref)
        cnt_ref[...] = jnp.zeros_like(cnt_ref)

    out2 = zp_ref[...] / (den2_ref[...] + 1e-16) + b2_ref[...][None, :]
    b = batch_ref[...]
    gids = jax.lax.broadcasted_iota(jnp.int32, (_BN, _G), 1)
    onehot = (b == gids).astype(jnp.float32)
    sums_ref[...] += jnp.dot(onehot.T, out2,
                             preferred_element_type=jnp.float32)
    cnt_ref[...] += jnp.sum(onehot, axis=0, keepdims=True).T


def _k3(zp, den2, b2, batch2d):
    grid = _N // _BN
    return pl.pallas_call(
        _k3_body,
        grid=(grid,),
        in_specs=[
            pl.BlockSpec((_BN, _C), lambda i: (i, 0)),
            pl.BlockSpec((_BN, 1), lambda i: (i, 0)),
            pl.BlockSpec((_C,), lambda i: (0,)),
            pl.BlockSpec((_BN, 1), lambda i: (i, 0)),
        ],
        out_specs=[
            pl.BlockSpec((_G, _C), lambda i: (0, 0)),
            pl.BlockSpec((_G, 1), lambda i: (0, 0)),
        ],
        out_shape=[
            jax.ShapeDtypeStruct((_G, _C), jnp.float32),
            jax.ShapeDtypeStruct((_G, 1), jnp.float32),
        ],
    )(zp, den2, b2, batch2d)


def _k4_body(sums_ref, cnt_ref, ed_ref, we_ref, be_ref,
             wf1_ref, bf1_ref, wf2_ref, bf2_ref, out_ref):
    pooled = sums_ref[...] / jnp.maximum(cnt_ref[...], 1.0)
    evo = jax.nn.relu(jnp.dot(ed_ref[...], we_ref[...],
                              preferred_element_type=jnp.float32)
                      + be_ref[...][None, :])
    comb = jnp.concatenate([pooled, evo], axis=1)
    hmid = jax.nn.relu(jnp.dot(comb, wf1_ref[...],
                               preferred_element_type=jnp.float32)
                       + bf1_ref[...][None, :])
    out_ref[...] = jnp.dot(hmid, wf2_ref[...],
                           preferred_element_type=jnp.float32) + bf2_ref[...][None, :]


def _k4(sums, cnt, ed2d, We, be, Wf1, bf1, Wf2, bf2):
    return pl.pallas_call(
        _k4_body,
        out_shape=jax.ShapeDtypeStruct((_G, _C), jnp.float32),
    )(sums, cnt, ed2d, We, be, Wf1, bf1, Wf2, bf2)


def kernel(x, evolutionary_distance, W1, a_src1, a_dst1, b1, W2, a_src2,
           a_dst2, b2, We, be, Wf1, bf1, Wf2, bf2, edge_index, batch):
    src = edge_index[0]
    dst = edge_index[1]

    s1, d1 = _k1(x, W1, a_src1, a_dst1)

    alpha = jax.nn.leaky_relu(s1[src] + d1[dst], negative_slope=0.2)
    amax = jax.ops.segment_max(alpha, dst, num_segments=_N)
    amax = jnp.where(jnp.isfinite(amax), amax, 0.0)
    ex = jnp.exp(alpha - amax[dst])
    den = jax.ops.segment_sum(ex, dst, num_segments=_N)
    xs = x[src]
    zx = jax.ops.segment_sum(
        (ex[:, :, None] * xs[:, None, :]).reshape(_E, _H * 4),
        dst, num_segments=_N)

    p, s2, d2 = _k2(zx, den, W1, b1, W2, a_src2, a_dst2)

    alpha2 = jax.nn.leaky_relu(s2[src, 0] + d2[dst, 0],
                               negative_slope=0.2)
    amax2 = jax.ops.segment_max(alpha2, dst, num_segments=_N)
    amax2 = jnp.where(jnp.isfinite(amax2), amax2, 0.0)
    ex2 = jnp.exp(alpha2 - amax2[dst])
    den2 = jax.ops.segment_sum(ex2, dst, num_segments=_N)
    zp = jax.ops.segment_sum(ex2[:, None] * p[src], dst,
                             num_segments=_N)

    sums, cnt = _k3(zp, den2[:, None], b2, batch[:, None].astype(jnp.int32))

    ed2d = evolutionary_distance[:, None]
    return _k4(sums, cnt, ed2d, We, be, Wf1, bf1, Wf2, bf2)


# speedup vs baseline: 4.6776x; 2.0092x over previous
import jax
import jax.numpy as jnp
from jax.experimental import pallas as pl

_N = 94208
_E = 524288
_G = 2048
_H = 4
_C = 64
_BN = 512


def _k1_body(x_ref, w1_ref, as_ref, ad_ref, s_ref, d_ref):
    x = x_ref[...]
    w1 = w1_ref[...]
    h = jnp.dot(x, w1, preferred_element_type=jnp.float32)
    h4 = h.reshape(_BN, _H, _C)
    s_ref[...] = jnp.sum(h4 * as_ref[...][None], axis=-1)
    d_ref[...] = jnp.sum(h4 * ad_ref[...][None], axis=-1)


def _k1(x, W1, a_src1, a_dst1):
    grid = _N // _BN
    return pl.pallas_call(
        _k1_body,
        grid=(grid,),
        in_specs=[
            pl.BlockSpec((_BN, 4), lambda i: (i, 0)),
            pl.BlockSpec((4, _H * _C), lambda i: (0, 0)),
            pl.BlockSpec((_H, _C), lambda i: (0, 0)),
            pl.BlockSpec((_H, _C), lambda i: (0, 0)),
        ],
        out_specs=[
            pl.BlockSpec((_BN, _H), lambda i: (i, 0)),
            pl.BlockSpec((_BN, _H), lambda i: (i, 0)),
        ],
        out_shape=[
            jax.ShapeDtypeStruct((_N, _H), jnp.float32),
            jax.ShapeDtypeStruct((_N, _H), jnp.float32),
        ],
    )(x, W1, a_src1, a_dst1)


def _k2_body(zx_ref, den_ref, w1_ref, b1_ref, w2_ref, as2_ref, ad2_ref,
             p_ref, s2_ref, d2_ref):
    zx = zx_ref[...]
    den = den_ref[...]
    w1 = w1_ref[...]
    outs = []
    for h in range(_H):
        zh = zx[:, h * 4:(h + 1) * 4] / (den[:, h:h + 1] + 1e-16)
        outs.append(jnp.dot(zh, w1[:, h * _C:(h + 1) * _C],
                            preferred_element_type=jnp.float32))
    h1 = jnp.concatenate(outs, axis=1) + b1_ref[...][None, :]
    g = jnp.where(h1 > 0, h1, jnp.exp(jnp.minimum(h1, 0.0)) - 1.0)
    p = jnp.dot(g, w2_ref[...], preferred_element_type=jnp.float32)
    p_ref[...] = p
    s2_ref[...] = jnp.sum(p * as2_ref[...], axis=-1, keepdims=True)
    d2_ref[...] = jnp.sum(p * ad2_ref[...], axis=-1, keepdims=True)


def _k2(zx, den, W1, b1, W2, a_src2, a_dst2):
    grid = _N // _BN
    return pl.pallas_call(
        _k2_body,
        grid=(grid,),
        in_specs=[
            pl.BlockSpec((_BN, _H * 4), lambda i: (i, 0)),
            pl.BlockSpec((_BN, _H), lambda i: (i, 0)),
            pl.BlockSpec((4, _H * _C), lambda i: (0, 0)),
            pl.BlockSpec((_H * _C,), lambda i: (0,)),
            pl.BlockSpec((_H * _C, _C), lambda i: (0, 0)),
            pl.BlockSpec((1, _C), lambda i: (0, 0)),
            pl.BlockSpec((1, _C), lambda i: (0, 0)),
        ],
        out_specs=[
            pl.BlockSpec((_BN, _C), lambda i: (i, 0)),
            pl.BlockSpec((_BN, 1), lambda i: (i, 0)),
            pl.BlockSpec((_BN, 1), lambda i: (i, 0)),
        ],
        out_shape=[
            jax.ShapeDtypeStruct((_N, _C), jnp.float32),
            jax.ShapeDtypeStruct((_N, 1), jnp.float32),
            jax.ShapeDtypeStruct((_N, 1), jnp.float32),
        ],
    )(zx, den, W1, b1, W2, a_src2, a_dst2)


_BE = 4096


def _ke1_body(s_ref, d_ref, xs_ref, m_ref, out_ref):
    alpha = jax.nn.leaky_relu(s_ref[...] + d_ref[...],
                              negative_slope=0.2)
    ex = jnp.exp(alpha - m_ref[...])
    z = (ex[:, :, None] * xs_ref[...][:, None, :]).reshape(_BE, 16)
    out_ref[...] = jnp.concatenate([ex, z], axis=1)


def _ke1(s1s, d1d, xs, m1):
    grid = _E // _BE
    return pl.pallas_call(
        _ke1_body,
        grid=(grid,),
        in_specs=[
            pl.BlockSpec((_BE, _H), lambda i: (i, 0)),
            pl.BlockSpec((_BE, _H), lambda i: (i, 0)),
            pl.BlockSpec((_BE, 4), lambda i: (i, 0)),
            pl.BlockSpec((1, _H), lambda i: (0, 0)),
        ],
        out_specs=pl.BlockSpec((_BE, 20), lambda i: (i, 0)),
        out_shape=jax.ShapeDtypeStruct((_E, 20), jnp.float32),
    )(s1s, d1d, xs, m1)


def _ke2_body(s_ref, d_ref, ps_ref, m_ref, out_ref):
    alpha = jax.nn.leaky_relu(s_ref[...] + d_ref[...],
                              negative_slope=0.2)
    ex = jnp.exp(alpha - m_ref[...])
    out_ref[...] = jnp.concatenate([ex, ex * ps_ref[...]], axis=1)


def _ke2(s2s, d2d, ps, m2):
    grid = _E // _BE
    return pl.pallas_call(
        _ke2_body,
        grid=(grid,),
        in_specs=[
            pl.BlockSpec((_BE, 1), lambda i: (i, 0)),
            pl.BlockSpec((_BE, 1), lambda i: (i, 0)),
            pl.BlockSpec((_BE, _C), lambda i: (i, 0)),
            pl.BlockSpec((1, 1), lambda i: (0, 0)),
        ],
        out_specs=pl.BlockSpec((_BE, 1 + _C), lambda i: (i, 0)),
        out_shape=jax.ShapeDtypeStruct((_E, 1 + _C), jnp.float32),
    )(s2s, d2d, ps, m2)


def _k3_body(zp_ref, den2_ref, b2_ref, batch_ref, sums_ref, cnt_ref):
    i = pl.program_id(0)

    @pl.when(i == 0)
    def _init():
        sums_ref[...] = jnp.zeros_like(sums_ref)
        cnt_ref[...] = jnp.zeros_like(cnt_ref)

    out2 = zp_ref[...] / (den2_ref[...] + 1e-16) + b2_ref[...][None, :]
    b = batch_ref[...]
    gids = jax.lax.broadcasted_iota(jnp.int32, (_BN, _G), 1)
    onehot = (b == gids).astype(jnp.float32)
    sums_ref[...] += jnp.dot(onehot.T, out2,
                             preferred_element_type=jnp.float32)
    cnt_ref[...] += jnp.sum(onehot, axis=0, keepdims=True).T


def _k3(zp, den2, b2, batch2d):
    grid = _N // _BN
    return pl.pallas_call(
        _k3_body,
        grid=(grid,),
        in_specs=[
            pl.BlockSpec((_BN, _C), lambda i: (i, 0)),
            pl.BlockSpec((_BN, 1), lambda i: (i, 0)),
            pl.BlockSpec((_C,), lambda i: (0,)),
            pl.BlockSpec((_BN, 1), lambda i: (i, 0)),
        ],
        out_specs=[
            pl.BlockSpec((_G, _C), lambda i: (0, 0)),
            pl.BlockSpec((_G, 1), lambda i: (0, 0)),
        ],
        out_shape=[
            jax.ShapeDtypeStruct((_G, _C), jnp.float32),
            jax.ShapeDtypeStruct((_G, 1), jnp.float32),
        ],
    )(zp, den2, b2, batch2d)


def _k4_body(sums_ref, cnt_ref, ed_ref, we_ref, be_ref,
             wf1_ref, bf1_ref, wf2_ref, bf2_ref, out_ref):
    pooled = sums_ref[...] / jnp.maximum(cnt_ref[...], 1.0)
    evo = jax.nn.relu(jnp.dot(ed_ref[...], we_ref[...],
                              preferred_element_type=jnp.float32)
                      + be_ref[...][None, :])
    comb = jnp.concatenate([pooled, evo], axis=1)
    hmid = jax.nn.relu(jnp.dot(comb, wf1_ref[...],
                               preferred_element_type=jnp.float32)
                       + bf1_ref[...][None, :])
    out_ref[...] = jnp.dot(hmid, wf2_ref[...],
                           preferred_element_type=jnp.float32) + bf2_ref[...][None, :]


def _k4(sums, cnt, ed2d, We, be, Wf1, bf1, Wf2, bf2):
    return pl.pallas_call(
        _k4_body,
        out_shape=jax.ShapeDtypeStruct((_G, _C), jnp.float32),
    )(sums, cnt, ed2d, We, be, Wf1, bf1, Wf2, bf2)


def kernel(x, evolutionary_distance, W1, a_src1, a_dst1, b1, W2, a_src2,
           a_dst2, b2, We, be, Wf1, bf1, Wf2, bf2, edge_index, batch):
    src = edge_index[0]
    dst = edge_index[1]

    s1, d1 = _k1(x, W1, a_src1, a_dst1)

    m1 = jax.nn.leaky_relu(jnp.max(s1, axis=0) + jnp.max(d1, axis=0),
                           negative_slope=0.2)[None, :]
    pay1 = _ke1(s1[src], d1[dst], x[src], m1)
    seg1 = jax.ops.segment_sum(pay1, dst, num_segments=_N)
    den = seg1[:, :_H]
    zx = seg1[:, _H:]

    p, s2, d2 = _k2(zx, den, W1, b1, W2, a_src2, a_dst2)

    m2 = jax.nn.leaky_relu(jnp.max(s2) + jnp.max(d2),
                           negative_slope=0.2)[None, None]
    pay2 = _ke2(s2[src], d2[dst], p[src], m2)
    seg2 = jax.ops.segment_sum(pay2, dst, num_segments=_N)
    den2 = seg2[:, 0]
    zp = seg2[:, 1:]

    sums, cnt = _k3(zp, den2[:, None], b2, batch[:, None].astype(jnp.int32))

    ed2d = evolutionary_distance[:, None]
    return _k4(sums, cnt, ed2d, We, be, Wf1, bf1, Wf2, bf2)


# speedup vs baseline: 6.6174x; 1.4147x over previous
import jax
import jax.numpy as jnp
from jax.experimental import pallas as pl

_N = 94208
_E = 524288
_G = 2048
_H = 4
_C = 64
_BN = 512


def _k1_body(x_ref, w1_ref, as_ref, ad_ref, s_ref, d_ref):
    x = x_ref[...]
    w1 = w1_ref[...]
    h = jnp.dot(x, w1, preferred_element_type=jnp.float32)
    h4 = h.reshape(_BN, _H, _C)
    s = jnp.sum(h4 * as_ref[...][None], axis=-1)
    s_ref[...] = jnp.concatenate([s, x], axis=1)
    d_ref[...] = jnp.sum(h4 * ad_ref[...][None], axis=-1)


def _k1(x, W1, a_src1, a_dst1):
    grid = _N // _BN
    return pl.pallas_call(
        _k1_body,
        grid=(grid,),
        in_specs=[
            pl.BlockSpec((_BN, 4), lambda i: (i, 0)),
            pl.BlockSpec((4, _H * _C), lambda i: (0, 0)),
            pl.BlockSpec((_H, _C), lambda i: (0, 0)),
            pl.BlockSpec((_H, _C), lambda i: (0, 0)),
        ],
        out_specs=[
            pl.BlockSpec((_BN, _H + 4), lambda i: (i, 0)),
            pl.BlockSpec((_BN, _H), lambda i: (i, 0)),
        ],
        out_shape=[
            jax.ShapeDtypeStruct((_N, _H + 4), jnp.float32),
            jax.ShapeDtypeStruct((_N, _H), jnp.float32),
        ],
    )(x, W1, a_src1, a_dst1)


def _k2_body(zx_ref, den_ref, w1_ref, b1_ref, w2_ref, as2_ref, ad2_ref,
             p_ref, d2_ref):
    zx = zx_ref[...]
    den = den_ref[...]
    w1 = w1_ref[...]
    outs = []
    for h in range(_H):
        zh = zx[:, h * 4:(h + 1) * 4] / (den[:, h:h + 1] + 1e-16)
        outs.append(jnp.dot(zh, w1[:, h * _C:(h + 1) * _C],
                            preferred_element_type=jnp.float32))
    h1 = jnp.concatenate(outs, axis=1) + b1_ref[...][None, :]
    g = jnp.where(h1 > 0, h1, jnp.exp(jnp.minimum(h1, 0.0)) - 1.0)
    p = jnp.dot(g, w2_ref[...], preferred_element_type=jnp.float32)
    s2 = jnp.sum(p * as2_ref[...], axis=-1, keepdims=True)
    p_ref[...] = jnp.concatenate([s2, p], axis=1)
    d2_ref[...] = jnp.sum(p * ad2_ref[...], axis=-1, keepdims=True)


def _k2(zx, den, W1, b1, W2, a_src2, a_dst2):
    grid = _N // _BN
    return pl.pallas_call(
        _k2_body,
        grid=(grid,),
        in_specs=[
            pl.BlockSpec((_BN, _H * 4), lambda i: (i, 0)),
            pl.BlockSpec((_BN, _H), lambda i: (i, 0)),
            pl.BlockSpec((4, _H * _C), lambda i: (0, 0)),
            pl.BlockSpec((_H * _C,), lambda i: (0,)),
            pl.BlockSpec((_H * _C, _C), lambda i: (0, 0)),
            pl.BlockSpec((1, _C), lambda i: (0, 0)),
            pl.BlockSpec((1, _C), lambda i: (0, 0)),
        ],
        out_specs=[
            pl.BlockSpec((_BN, 1 + _C), lambda i: (i, 0)),
            pl.BlockSpec((_BN, 1), lambda i: (i, 0)),
        ],
        out_shape=[
            jax.ShapeDtypeStruct((_N, 1 + _C), jnp.float32),
            jax.ShapeDtypeStruct((_N, 1), jnp.float32),
        ],
    )(zx, den, W1, b1, W2, a_src2, a_dst2)


_BE = 4096


def _ke1_body(ts_ref, d_ref, m_ref, out_ref):
    ts = ts_ref[...]
    alpha = jax.nn.leaky_relu(ts[:, :_H] + d_ref[...],
                              negative_slope=0.2)
    ex = jnp.exp(alpha - m_ref[...])
    z = (ex[:, :, None] * ts[:, None, _H:]).reshape(_BE, 16)
    out_ref[...] = jnp.concatenate([ex, z], axis=1)


def _ke1(t1s, d1d, m1):
    grid = _E // _BE
    return pl.pallas_call(
        _ke1_body,
        grid=(grid,),
        in_specs=[
            pl.BlockSpec((_BE, _H + 4), lambda i: (i, 0)),
            pl.BlockSpec((_BE, _H), lambda i: (i, 0)),
            pl.BlockSpec((1, _H), lambda i: (0, 0)),
        ],
        out_specs=pl.BlockSpec((_BE, 20), lambda i: (i, 0)),
        out_shape=jax.ShapeDtypeStruct((_E, 20), jnp.float32),
    )(t1s, d1d, m1)


def _ke2_body(ts_ref, d_ref, m_ref, out_ref):
    ts = ts_ref[...]
    alpha = jax.nn.leaky_relu(ts[:, :1] + d_ref[...],
                              negative_slope=0.2)
    ex = jnp.exp(alpha - m_ref[...])
    out_ref[...] = jnp.concatenate([ex, ex * ts[:, 1:]], axis=1)


def _ke2(t2s, d2d, m2):
    grid = _E // _BE
    return pl.pallas_call(
        _ke2_body,
        grid=(grid,),
        in_specs=[
            pl.BlockSpec((_BE, 1 + _C), lambda i: (i, 0)),
            pl.BlockSpec((_BE, 1), lambda i: (i, 0)),
            pl.BlockSpec((1, 1), lambda i: (0, 0)),
        ],
        out_specs=pl.BlockSpec((_BE, 1 + _C), lambda i: (i, 0)),
        out_shape=jax.ShapeDtypeStruct((_E, 1 + _C), jnp.float32),
    )(t2s, d2d, m2)


def _k3_body(zp_ref, den2_ref, b2_ref, batch_ref, sums_ref, cnt_ref):
    i = pl.program_id(0)

    @pl.when(i == 0)
    def _init():
        sums_ref[...] = jnp.zeros_like(sums_ref)
        cnt_ref[...] = jnp.zeros_like(cnt_ref)

    out2 = zp_ref[...] / (den2_ref[...] + 1e-16) + b2_ref[...][None, :]
    b = batch_ref[...]
    gids = jax.lax.broadcasted_iota(jnp.int32, (_BN, _G), 1)
    onehot = (b == gids).astype(jnp.float32)
    sums_ref[...] += jnp.dot(onehot.T, out2,
                             preferred_element_type=jnp.float32)
    cnt_ref[...] += jnp.sum(onehot, axis=0, keepdims=True).T


def _k3(zp, den2, b2, batch2d):
    grid = _N // _BN
    return pl.pallas_call(
        _k3_body,
        grid=(grid,),
        in_specs=[
            pl.BlockSpec((_BN, _C), lambda i: (i, 0)),
            pl.BlockSpec((_BN, 1), lambda i: (i, 0)),
            pl.BlockSpec((_C,), lambda i: (0,)),
            pl.BlockSpec((_BN, 1), lambda i: (i, 0)),
        ],
        out_specs=[
            pl.BlockSpec((_G, _C), lambda i: (0, 0)),
            pl.BlockSpec((_G, 1), lambda i: (0, 0)),
        ],
        out_shape=[
            jax.ShapeDtypeStruct((_G, _C), jnp.float32),
            jax.ShapeDtypeStruct((_G, 1), jnp.float32),
        ],
    )(zp, den2, b2, batch2d)


def _k4_body(sums_ref, cnt_ref, ed_ref, we_ref, be_ref,
             wf1_ref, bf1_ref, wf2_ref, bf2_ref, out_ref):
    pooled = sums_ref[...] / jnp.maximum(cnt_ref[...], 1.0)
    evo = jax.nn.relu(jnp.dot(ed_ref[...], we_ref[...],
                              preferred_element_type=jnp.float32)
                      + be_ref[...][None, :])
    comb = jnp.concatenate([pooled, evo], axis=1)
    hmid = jax.nn.relu(jnp.dot(comb, wf1_ref[...],
                               preferred_element_type=jnp.float32)
                       + bf1_ref[...][None, :])
    out_ref[...] = jnp.dot(hmid, wf2_ref[...],
                           preferred_element_type=jnp.float32) + bf2_ref[...][None, :]


def _k4(sums, cnt, ed2d, We, be, Wf1, bf1, Wf2, bf2):
    return pl.pallas_call(
        _k4_body,
        out_shape=jax.ShapeDtypeStruct((_G, _C), jnp.float32),
    )(sums, cnt, ed2d, We, be, Wf1, bf1, Wf2, bf2)


def kernel(x, evolutionary_distance, W1, a_src1, a_dst1, b1, W2, a_src2,
           a_dst2, b2, We, be, Wf1, bf1, Wf2, bf2, edge_index, batch):
    src = edge_index[0]
    dst = edge_index[1]

    t1, d1 = _k1(x, W1, a_src1, a_dst1)

    m1 = jax.nn.leaky_relu(jnp.max(t1[:, :_H], axis=0) + jnp.max(d1, axis=0),
                           negative_slope=0.2)[None, :]
    pay1 = _ke1(t1[src], d1[dst], m1)
    seg1 = jax.ops.segment_sum(pay1, dst, num_segments=_N)
    den = seg1[:, :_H]
    zx = seg1[:, _H:]

    t2, d2 = _k2(zx, den, W1, b1, W2, a_src2, a_dst2)

    m2 = jax.nn.leaky_relu(jnp.max(t2[:, 0]) + jnp.max(d2),
                           negative_slope=0.2)[None, None]
    pay2 = _ke2(t2[src], d2[dst], m2)
    seg2 = jax.ops.segment_sum(pay2, dst, num_segments=_N)
    den2 = seg2[:, 0]
    zp = seg2[:, 1:]

    sums, cnt = _k3(zp, den2[:, None], b2, batch[:, None].astype(jnp.int32))

    ed2d = evolutionary_distance[:, None]
    return _k4(sums, cnt, ed2d, We, be, Wf1, bf1, Wf2, bf2)
